# baseline (speedup 1.0000x reference)
import functools

import jax
import jax.numpy as jnp
from jax import lax
from jax.experimental import pallas as pl
from jax.experimental.pallas import tpu as pltpu
from jax.experimental.pallas import tpu_sc as plsc

N = 2048
D = 256
NGL = 3
Bn, Rn, Ln = 8, 32, 64
E = Rn * Ln
NC, NS = 2, 16
EPT = E // NS
RPW = (Bn * Rn) // (NC * NS)
BPC = Bn // NC


def _lrelu(x):
    return jnp.where(x >= 0, x, 0.01 * x)


def _d(a, b):
    return jnp.dot(a, b, preferred_element_type=jnp.float32)


def _dot_oh(m, x):
    x_hi = x.astype(jnp.bfloat16).astype(jnp.float32)
    return _d(m, x_hi) + _d(m, x - x_hi)


DCH = 16
NDCH = D // DCH
GCH = 128
CPT = NDCH * Bn // (NC * NS)


def _make_sc_agg(with_routes):
    mesh = plsc.VectorSubcoreMesh(core_axis_name="c", subcore_axis_name="s",
                                  num_cores=NC, num_subcores=NS)
    out_type = [jax.ShapeDtypeStruct((Bn * NDCH * N * DCH,), jnp.float32)]
    if with_routes:
        out_type.append(jax.ShapeDtypeStruct((Bn * Rn * N,), jnp.float32))
    EH = E // 2
    EPH = EH // NS
    N2 = N // 2
    NPH = N2 // NS
    scratch = [
        pltpu.VMEM((EPH, D), jnp.float32),
        pltpu.VMEM((E * DCH,), jnp.float32),
        pltpu.VMEM((N * DCH,), jnp.float32),
        pltpu.VMEM((NDCH * EPH * DCH,), jnp.float32),
        pltpu.VMEM((EPH,), jnp.int32),
        pltpu.VMEM((E,), jnp.int32),
        pltpu.VMEM_SHARED((NDCH * EH * DCH,), jnp.float32),
        pltpu.SemaphoreType.DMA,
    ]
    if with_routes:
        scratch += [
            pltpu.VMEM((N,), jnp.float32),
            pltpu.VMEM((2 * Ln,), jnp.int32),
        ]

    def body(h_hbm, src_hbm, dst_hbm, ridx_hbm, agg_hbm, rcnt_hbm,
             rows, crows, agf, stg, siv, div, planes, sem,
             cnt=None, riv=None):
        c = lax.axis_index("c")
        s = lax.axis_index("s")
        wid = c * NS + s
        zero16 = jnp.zeros((16,), jnp.float32)
        iota16 = lax.iota(jnp.int32, 16)

        if with_routes:
            ones16 = jnp.ones((16,), jnp.float32)

            @plsc.parallel_loop(0, N // 16, unroll=8)
            def zero_cnt(i):
                cnt[pl.ds(i * 16, 16)] = zero16
            for k in range(RPW):
                rid = wid * RPW + k
                pltpu.sync_copy(ridx_hbm.at[pl.ds(rid * 2 * Ln, 2 * Ln)], riv)
                for g in range(2 * Ln // 16):
                    iv = riv[pl.ds(g * 16, 16)]
                    plsc.addupdate_scatter(cnt, [iv], ones16)
                pltpu.sync_copy(cnt, rcnt_hbm.at[pl.ds(rid * N, N)])
                for g in range(2 * Ln // 16):
                    iv = riv[pl.ds(g * 16, 16)]
                    plsc.store_scatter(cnt, [iv], zero16)

        iotaL = iota16 * DCH

        def scenario(w, carry):
            b = c * BPC + w
            for p in range(2):
                pltpu.sync_copy(
                    src_hbm.at[pl.ds(b * E + p * EH + s * EPH, EPH)], siv)
                pltpu.async_copy(h_hbm.at[siv], rows, sem).wait()

                @plsc.parallel_loop(0, EPH, unroll=2)
                def tr_out(r):
                    for k in range(NDCH):
                        stg[pl.ds((k * EPH + r) * DCH, DCH)] = \
                            rows[r, pl.ds(k * DCH, DCH)]
                dsc = [pltpu.async_copy(
                    stg.at[pl.ds(k * EPH * DCH, EPH * DCH)],
                    planes.at[pl.ds((k * EH + s * EPH) * DCH, EPH * DCH)],
                    sem) for k in range(NDCH)]
                for dd in dsc:
                    dd.wait()
                plsc.subcore_barrier()
                pltpu.sync_copy(planes.at[pl.ds(s * EH * DCH, EH * DCH)],
                                crows.at[pl.ds(p * EH * DCH, EH * DCH)])
                plsc.subcore_barrier()

            pltpu.sync_copy(dst_hbm.at[pl.ds(b * E, E)], div)

            @plsc.parallel_loop(0, N * DCH // 16, unroll=8)
            def zero_agg(i):
                agf[pl.ds(i * 16, 16)] = zero16

            @plsc.parallel_loop(0, E // 16, unroll=2)
            def accum(i):
                dv = div[pl.ds(i * 16, 16)] * DCH
                base = i * (16 * DCH) + iotaL
                for kk in range(DCH):
                    col = plsc.load_gather(crows, [base + kk])
                    plsc.addupdate_scatter(agf, [dv + kk], col)

            pltpu.sync_copy(
                agf, agg_hbm.at[pl.ds((b * NDCH + s) * N * DCH, N * DCH)])
            return carry
        lax.fori_loop(0, BPC, scenario, 0)

    if not with_routes:
        def body_nr(h_hbm, src_hbm, dst_hbm, ridx_hbm, agg_hbm,
                    rows, crows, agf, stg, siv, div, planes, sem):
            body(h_hbm, src_hbm, dst_hbm, ridx_hbm, agg_hbm, None,
                 rows, crows, agf, stg, siv, div, planes, sem)
        fn = body_nr
    else:
        fn = body
    return pl.kernel(fn, out_type=out_type, mesh=mesh, scratch_types=scratch,
                     compiler_params=pltpu.CompilerParams(
                         needs_layout_passes=False))


_sc_layer0 = _make_sc_agg(with_routes=True)
_sc_layer = _make_sc_agg(with_routes=False)


def _embed_body(wemb_ref, bemb_ref, h0_ref):
    h0_ref[...] = (wemb_ref[...].astype(jnp.bfloat16).astype(jnp.float32)
                   + bemb_ref[...])


def _recombine(a):
    return jnp.transpose(a, (1, 0, 2)).reshape(N, D)


def _layer0_body(h0_ref, agg_ref, w_ref, b_ref, out_ref):
    h = h0_ref[...] + _recombine(agg_ref[0])
    out_ref[...] = _lrelu(_d(h, w_ref[...]) + b_ref[...]).reshape(1, N, D)


def _layer_body(hp_ref, agg_ref, w_ref, b_ref, out_ref):
    h = hp_ref[0] + _recombine(agg_ref[0])
    out_ref[...] = _lrelu(_d(h, w_ref[...]) + b_ref[...]).reshape(1, N, D)


def _final_body(h_ref, rcnt_ref,
                ghW1_ref, ghb1_ref, ghW2_ref, ghb2_ref, ghW3_ref, ghb3_ref,
                rhW1_ref, rhb1_ref, rhW2_ref, rhb2_ref, rhW3_ref, rhb3_ref,
                gout_ref, rout_ref):
    f32 = jnp.float32
    h = h_ref[0]
    rc = rcnt_ref[0]
    rmask = (rc > 0).astype(f32)
    counts = jnp.sum(rmask, axis=1, keepdims=True)
    ucnt = jnp.sum(rc, axis=0, keepdims=True)
    used = (ucnt > 0).astype(f32)
    gdesc = _dot_oh(used, h) / 1000.0
    rdesc = _d(rmask, h) / counts

    gh = _lrelu(_d(gdesc, ghW1_ref[...]) + ghb1_ref[...])
    gh = _lrelu(_d(gh, ghW2_ref[...]) + ghb2_ref[...])
    gpred = _d(gh, ghW3_ref[...]) + ghb3_ref[...]

    tiled = jnp.broadcast_to(_lrelu(gdesc), (Rn, D))
    rin = jnp.concatenate([tiled, rdesc], axis=1)
    rh = _lrelu(_d(rin, rhW1_ref[...]) + rhb1_ref[...])
    rh = _lrelu(_d(rh, rhW2_ref[...]) + rhb2_ref[...])
    rpred = _d(rh, rhW3_ref[...]) + rhb3_ref[...]

    gout_ref[...] = gpred.reshape(1, 1, 1)
    rout_ref[...] = rpred.reshape(1, Rn, 1)


def _rep2(shape):
    return pl.BlockSpec(shape, lambda b: (0, 0))


def _layer_call(body, h_spec, h_arr, agg, w, bias):
    return pl.pallas_call(
        body,
        grid=(Bn,),
        in_specs=[h_spec,
                  pl.BlockSpec((1, NDCH, N, DCH), lambda b: (b, 0, 0, 0)),
                  _rep2((D, D)), _rep2((1, D))],
        out_specs=pl.BlockSpec((1, N, D), lambda b: (b, 0, 0)),
        out_shape=jax.ShapeDtypeStruct((Bn, N, D), jnp.float32),
        compiler_params=pltpu.CompilerParams(
            dimension_semantics=("arbitrary",)),
    )(h_arr, agg, w, bias)


def kernel(env_rep, batch_route_idxs, Wemb, bemb, gcn_W, gcn_b,
           gh_W1, gh_b1, gh_W2, gh_b2, gh_W3, gh_b3,
           rh_W1, rh_b1, rh_W2, rh_b2, rh_W3, rh_b3):
    f32 = jnp.float32
    src = batch_route_idxs[:, :, 0, :].reshape(Bn, E)
    dst = batch_route_idxs[:, :, 1, :].reshape(Bn, E)
    src1 = src.reshape(Bn * E)
    dst1 = dst.reshape(Bn * E)
    srcoff1 = (src + jnp.arange(Bn, dtype=jnp.int32)[:, None] * N).reshape(Bn * E)
    ridx1 = batch_route_idxs.reshape(Bn * Rn * 2 * Ln)

    h0 = pl.pallas_call(
        _embed_body,
        out_shape=jax.ShapeDtypeStruct((N, D), f32),
    )(Wemb, bemb.reshape(1, D))

    agg0, rcnt_flat = _sc_layer0(h0, src1, dst1, ridx1)
    h = _layer_call(_layer0_body, _rep2((N, D)), h0,
                    agg0.reshape(Bn, NDCH, N, DCH), gcn_W[0],
                    gcn_b[0].reshape(1, D))
    for i in range(1, NGL):
        (agg,) = _sc_layer(h.reshape(Bn * N, D), srcoff1, dst1, ridx1)
        h = _layer_call(_layer_body,
                        pl.BlockSpec((1, N, D), lambda b: (b, 0, 0)), h,
                        agg.reshape(Bn, NDCH, N, DCH), gcn_W[i],
                        gcn_b[i].reshape(1, D))

    rcnt = rcnt_flat.reshape(Bn, Rn, N)
    gpred, rpred = pl.pallas_call(
        _final_body,
        grid=(Bn,),
        in_specs=[pl.BlockSpec((1, N, D), lambda b: (b, 0, 0)),
                  pl.BlockSpec((1, Rn, N), lambda b: (b, 0, 0)),
                  _rep2((D, D)), _rep2((1, D)),
                  _rep2((D, D)), _rep2((1, D)),
                  _rep2((D, 1)), _rep2((1, 1)),
                  _rep2((2 * D, D)), _rep2((1, D)),
                  _rep2((D, D)), _rep2((1, D)),
                  _rep2((D, 1)), _rep2((1, 1))],
        out_specs=[pl.BlockSpec((1, 1, 1), lambda b: (b, 0, 0)),
                   pl.BlockSpec((1, Rn, 1), lambda b: (b, 0, 0))],
        out_shape=[jax.ShapeDtypeStruct((Bn, 1, 1), f32),
                   jax.ShapeDtypeStruct((Bn, Rn, 1), f32)],
        compiler_params=pltpu.CompilerParams(
            dimension_semantics=("arbitrary",)),
    )(h, rcnt,
      gh_W1, gh_b1.reshape(1, D), gh_W2, gh_b2.reshape(1, D),
      gh_W3, gh_b3.reshape(1, 1),
      rh_W1, rh_b1.reshape(1, D), rh_W2, rh_b2.reshape(1, D),
      rh_W3, rh_b3.reshape(1, 1))
    return gpred.reshape(Bn, 1), rpred


# speedup vs baseline: 1.5797x; 1.5797x over previous
import functools

import jax
import jax.numpy as jnp
from jax import lax
from jax.experimental import pallas as pl
from jax.experimental.pallas import tpu as pltpu
from jax.experimental.pallas import tpu_sc as plsc

N = 2048
D = 256
NGL = 3
Bn, Rn, Ln = 8, 32, 64
E = Rn * Ln
NC, NS = 2, 16
EPT = E // NS
RPW = (Bn * Rn) // (NC * NS)
BPC = Bn // NC


def _lrelu(x):
    return jnp.where(x >= 0, x, 0.01 * x)


def _d(a, b):
    return jnp.dot(a, b, preferred_element_type=jnp.float32)


def _dot_oh(m, x):
    x_hi = x.astype(jnp.bfloat16).astype(jnp.float32)
    return _d(m, x_hi) + _d(m, x - x_hi)


DCH = 16
NDCH = D // DCH
GCH = 128
CPT = NDCH * Bn // (NC * NS)


def _make_sc_agg(with_routes):
    mesh = plsc.VectorSubcoreMesh(core_axis_name="c", subcore_axis_name="s",
                                  num_cores=NC, num_subcores=NS)
    out_type = [jax.ShapeDtypeStruct((Bn * N, D), jnp.float32)]
    if with_routes:
        out_type.append(jax.ShapeDtypeStruct((Bn * Rn * N,), jnp.float32))
    EH = E // 2
    EPH = EH // NS
    N2 = N // 2
    NPH = N2 // NS
    scratch = [
        pltpu.VMEM((EPH, D), jnp.float32),
        pltpu.VMEM((E * DCH,), jnp.float32),
        pltpu.VMEM((N * DCH,), jnp.float32),
        pltpu.VMEM((NDCH * EPH * DCH,), jnp.float32),
        pltpu.VMEM((EPH,), jnp.int32),
        pltpu.VMEM((E,), jnp.int32),
        pltpu.VMEM_SHARED((NDCH * EH * DCH,), jnp.float32),
        pltpu.SemaphoreType.DMA,
    ]
    if with_routes:
        scratch += [
            pltpu.VMEM((N,), jnp.float32),
            pltpu.VMEM((2 * Ln,), jnp.int32),
        ]

    def body(h_hbm, src_hbm, dst_hbm, ridx_hbm, agg_hbm, rcnt_hbm,
             rows, crows, agf, stg, siv, div, planes, sem,
             cnt=None, riv=None):
        c = lax.axis_index("c")
        s = lax.axis_index("s")
        wid = c * NS + s
        zero16 = jnp.zeros((16,), jnp.float32)
        iota16 = lax.iota(jnp.int32, 16)

        if with_routes:
            ones16 = jnp.ones((16,), jnp.float32)

            @plsc.parallel_loop(0, N // 16, unroll=8)
            def zero_cnt(i):
                cnt[pl.ds(i * 16, 16)] = zero16
            for k in range(RPW):
                rid = wid * RPW + k
                pltpu.sync_copy(ridx_hbm.at[pl.ds(rid * 2 * Ln, 2 * Ln)], riv)
                for g in range(2 * Ln // 16):
                    iv = riv[pl.ds(g * 16, 16)]
                    plsc.addupdate_scatter(cnt, [iv], ones16)
                pltpu.sync_copy(cnt, rcnt_hbm.at[pl.ds(rid * N, N)])
                for g in range(2 * Ln // 16):
                    iv = riv[pl.ds(g * 16, 16)]
                    plsc.store_scatter(cnt, [iv], zero16)

        iotaL = iota16 * DCH

        def scenario(w, carry):
            b = c * BPC + w
            for p in range(2):
                pltpu.sync_copy(
                    src_hbm.at[pl.ds(b * E + p * EH + s * EPH, EPH)], siv)
                pltpu.async_copy(h_hbm.at[siv], rows, sem).wait()

                @plsc.parallel_loop(0, EPH, unroll=2)
                def tr_out(r):
                    for k in range(NDCH):
                        stg[pl.ds((k * EPH + r) * DCH, DCH)] = \
                            rows[r, pl.ds(k * DCH, DCH)]
                dsc = [pltpu.async_copy(
                    stg.at[pl.ds(k * EPH * DCH, EPH * DCH)],
                    planes.at[pl.ds((k * EH + s * EPH) * DCH, EPH * DCH)],
                    sem) for k in range(NDCH)]
                for dd in dsc:
                    dd.wait()
                plsc.subcore_barrier()
                pltpu.sync_copy(planes.at[pl.ds(s * EH * DCH, EH * DCH)],
                                crows.at[pl.ds(p * EH * DCH, EH * DCH)])
                plsc.subcore_barrier()

            pltpu.sync_copy(dst_hbm.at[pl.ds(b * E, E)], div)

            @plsc.parallel_loop(0, N * DCH // 16, unroll=8)
            def zero_agg(i):
                agf[pl.ds(i * 16, 16)] = zero16

            @plsc.parallel_loop(0, E // 16, unroll=4)
            def accum(i):
                dv = div[pl.ds(i * 16, 16)] * DCH
                base = i * (16 * DCH) + iotaL
                for kk in range(DCH):
                    col = plsc.load_gather(crows, [base + kk])
                    plsc.addupdate_scatter(agf, [dv + kk], col)

            for q in range(2):
                pltpu.sync_copy(agf.at[pl.ds(q * N2 * DCH, N2 * DCH)],
                                planes.at[pl.ds(s * N2 * DCH, N2 * DCH)])
                plsc.subcore_barrier()
                dsc = [pltpu.async_copy(
                    planes.at[pl.ds((k * N2 + s * NPH) * DCH, NPH * DCH)],
                    stg.at[pl.ds(k * NPH * DCH, NPH * DCH)],
                    sem) for k in range(NDCH)]
                for dd in dsc:
                    dd.wait()

                @plsc.parallel_loop(0, NPH, unroll=2)
                def tr_in(r):
                    for k in range(NDCH):
                        rows[r, pl.ds(k * DCH, DCH)] = \
                            stg[pl.ds((k * NPH + r) * DCH, DCH)]
                pltpu.sync_copy(
                    rows,
                    agg_hbm.at[pl.ds(b * N + q * N2 + s * NPH, NPH)])
                plsc.subcore_barrier()
            return carry
        lax.fori_loop(0, BPC, scenario, 0)

    if not with_routes:
        def body_nr(h_hbm, src_hbm, dst_hbm, ridx_hbm, agg_hbm,
                    rows, crows, agf, stg, siv, div, planes, sem):
            body(h_hbm, src_hbm, dst_hbm, ridx_hbm, agg_hbm, None,
                 rows, crows, agf, stg, siv, div, planes, sem)
        fn = body_nr
    else:
        fn = body
    return pl.kernel(fn, out_type=out_type, mesh=mesh, scratch_types=scratch,
                     compiler_params=pltpu.CompilerParams(
                         needs_layout_passes=False))


_sc_layer0 = _make_sc_agg(with_routes=True)
_sc_layer = _make_sc_agg(with_routes=False)


def _embed_body(wemb_ref, bemb_ref, h0_ref):
    h0_ref[...] = (wemb_ref[...].astype(jnp.bfloat16).astype(jnp.float32)
                   + bemb_ref[...])


def _layer0_body(h0_ref, agg_ref, w_ref, b_ref, out_ref):
    h = h0_ref[...] + agg_ref[0]
    out_ref[...] = _lrelu(_d(h, w_ref[...]) + b_ref[...]).reshape(1, N, D)


def _layer_body(hp_ref, agg_ref, w_ref, b_ref, out_ref):
    h = hp_ref[0] + agg_ref[0]
    out_ref[...] = _lrelu(_d(h, w_ref[...]) + b_ref[...]).reshape(1, N, D)


def _final_body(h_ref, rcnt_ref,
                ghW1_ref, ghb1_ref, ghW2_ref, ghb2_ref, ghW3_ref, ghb3_ref,
                rhW1_ref, rhb1_ref, rhW2_ref, rhb2_ref, rhW3_ref, rhb3_ref,
                gout_ref, rout_ref):
    f32 = jnp.float32
    h = h_ref[0]
    rc = rcnt_ref[0]
    rmask = (rc > 0).astype(f32)
    counts = jnp.sum(rmask, axis=1, keepdims=True)
    ucnt = jnp.sum(rc, axis=0, keepdims=True)
    used = (ucnt > 0).astype(f32)
    gdesc = _dot_oh(used, h) / 1000.0
    rdesc = _d(rmask, h) / counts

    gh = _lrelu(_d(gdesc, ghW1_ref[...]) + ghb1_ref[...])
    gh = _lrelu(_d(gh, ghW2_ref[...]) + ghb2_ref[...])
    gpred = _d(gh, ghW3_ref[...]) + ghb3_ref[...]

    tiled = jnp.broadcast_to(_lrelu(gdesc), (Rn, D))
    rin = jnp.concatenate([tiled, rdesc], axis=1)
    rh = _lrelu(_d(rin, rhW1_ref[...]) + rhb1_ref[...])
    rh = _lrelu(_d(rh, rhW2_ref[...]) + rhb2_ref[...])
    rpred = _d(rh, rhW3_ref[...]) + rhb3_ref[...]

    gout_ref[...] = gpred.reshape(1, 1, 1)
    rout_ref[...] = rpred.reshape(1, Rn, 1)


def _rep2(shape):
    return pl.BlockSpec(shape, lambda b: (0, 0))


def _layer_call(body, h_spec, h_arr, agg, w, bias):
    return pl.pallas_call(
        body,
        grid=(Bn,),
        in_specs=[h_spec,
                  pl.BlockSpec((1, N, D), lambda b: (b, 0, 0)),
                  _rep2((D, D)), _rep2((1, D))],
        out_specs=pl.BlockSpec((1, N, D), lambda b: (b, 0, 0)),
        out_shape=jax.ShapeDtypeStruct((Bn, N, D), jnp.float32),
        compiler_params=pltpu.CompilerParams(
            dimension_semantics=("arbitrary",)),
    )(h_arr, agg, w, bias)


def kernel(env_rep, batch_route_idxs, Wemb, bemb, gcn_W, gcn_b,
           gh_W1, gh_b1, gh_W2, gh_b2, gh_W3, gh_b3,
           rh_W1, rh_b1, rh_W2, rh_b2, rh_W3, rh_b3):
    f32 = jnp.float32
    src = batch_route_idxs[:, :, 0, :].reshape(Bn, E)
    dst = batch_route_idxs[:, :, 1, :].reshape(Bn, E)
    src1 = src.reshape(Bn * E)
    dst1 = dst.reshape(Bn * E)
    srcoff1 = (src + jnp.arange(Bn, dtype=jnp.int32)[:, None] * N).reshape(Bn * E)
    ridx1 = batch_route_idxs.reshape(Bn * Rn * 2 * Ln)

    h0 = pl.pallas_call(
        _embed_body,
        out_shape=jax.ShapeDtypeStruct((N, D), f32),
    )(Wemb, bemb.reshape(1, D))

    agg0, rcnt_flat = _sc_layer0(h0, src1, dst1, ridx1)
    h = _layer_call(_layer0_body, _rep2((N, D)), h0,
                    agg0.reshape(Bn, N, D), gcn_W[0], gcn_b[0].reshape(1, D))
    for i in range(1, NGL):
        (agg,) = _sc_layer(h.reshape(Bn * N, D), srcoff1, dst1, ridx1)
        h = _layer_call(_layer_body,
                        pl.BlockSpec((1, N, D), lambda b: (b, 0, 0)), h,
                        agg.reshape(Bn, N, D), gcn_W[i],
                        gcn_b[i].reshape(1, D))

    rcnt = rcnt_flat.reshape(Bn, Rn, N)
    gpred, rpred = pl.pallas_call(
        _final_body,
        grid=(Bn,),
        in_specs=[pl.BlockSpec((1, N, D), lambda b: (b, 0, 0)),
                  pl.BlockSpec((1, Rn, N), lambda b: (b, 0, 0)),
                  _rep2((D, D)), _rep2((1, D)),
                  _rep2((D, D)), _rep2((1, D)),
                  _rep2((D, 1)), _rep2((1, 1)),
                  _rep2((2 * D, D)), _rep2((1, D)),
                  _rep2((D, D)), _rep2((1, D)),
                  _rep2((D, 1)), _rep2((1, 1))],
        out_specs=[pl.BlockSpec((1, 1, 1), lambda b: (b, 0, 0)),
                   pl.BlockSpec((1, Rn, 1), lambda b: (b, 0, 0))],
        out_shape=[jax.ShapeDtypeStruct((Bn, 1, 1), f32),
                   jax.ShapeDtypeStruct((Bn, Rn, 1), f32)],
        compiler_params=pltpu.CompilerParams(
            dimension_semantics=("arbitrary",)),
    )(h, rcnt,
      gh_W1, gh_b1.reshape(1, D), gh_W2, gh_b2.reshape(1, D),
      gh_W3, gh_b3.reshape(1, 1),
      rh_W1, rh_b1.reshape(1, D), rh_W2, rh_b2.reshape(1, D),
      rh_W3, rh_b3.reshape(1, 1))
    return gpred.reshape(Bn, 1), rpred


# speedup vs baseline: 1.7157x; 1.0861x over previous
import functools

import jax
import jax.numpy as jnp
from jax import lax
from jax.experimental import pallas as pl
from jax.experimental.pallas import tpu as pltpu
from jax.experimental.pallas import tpu_sc as plsc

N = 2048
D = 256
NGL = 3
Bn, Rn, Ln = 8, 32, 64
E = Rn * Ln
NC, NS = 2, 16
EPT = E // NS
RPW = (Bn * Rn) // (NC * NS)
BPC = Bn // NC


def _lrelu(x):
    return jnp.where(x >= 0, x, 0.01 * x)


def _d(a, b):
    return jnp.dot(a, b, preferred_element_type=jnp.float32)


def _dot_oh(m, x):
    x_hi = x.astype(jnp.bfloat16).astype(jnp.float32)
    return _d(m, x_hi) + _d(m, x - x_hi)


DCH = 16
NDCH = D // DCH
GCH = 128
CPT = NDCH * Bn // (NC * NS)


def _make_sc_agg(with_routes):
    mesh = plsc.VectorSubcoreMesh(core_axis_name="c", subcore_axis_name="s",
                                  num_cores=NC, num_subcores=NS)
    out_type = [jax.ShapeDtypeStruct((Bn * N, D), jnp.float32)]
    if with_routes:
        out_type.append(jax.ShapeDtypeStruct((Bn * Rn * N,), jnp.float32))
    EH = E // 2
    EPH = EH // NS
    N2 = N // 2
    NPH = N2 // NS
    scratch = [
        pltpu.VMEM((EPH, D), jnp.float32),
        pltpu.VMEM((E * DCH,), jnp.float32),
        pltpu.VMEM((N * DCH,), jnp.float32),
        pltpu.VMEM((NDCH * EPH * DCH,), jnp.float32),
        pltpu.VMEM((EPH,), jnp.int32),
        pltpu.VMEM((E,), jnp.int32),
        pltpu.VMEM_SHARED((NDCH * EH * DCH,), jnp.float32),
        pltpu.SemaphoreType.DMA,
    ]
    if with_routes:
        scratch += [
            pltpu.VMEM((N,), jnp.float32),
            pltpu.VMEM((2 * Ln,), jnp.int32),
        ]

    def body(h_hbm, src_hbm, dst_hbm, ridx_hbm, agg_hbm, rcnt_hbm,
             rows, crows, agf, stg, siv, div, planes, sem,
             cnt=None, riv=None):
        c = lax.axis_index("c")
        s = lax.axis_index("s")
        wid = c * NS + s
        zero16 = jnp.zeros((16,), jnp.float32)
        iota16 = lax.iota(jnp.int32, 16)

        if with_routes:
            ones16 = jnp.ones((16,), jnp.float32)

            @plsc.parallel_loop(0, N // 16, unroll=8)
            def zero_cnt(i):
                cnt[pl.ds(i * 16, 16)] = zero16
            for k in range(RPW):
                rid = wid * RPW + k
                pltpu.sync_copy(ridx_hbm.at[pl.ds(rid * 2 * Ln, 2 * Ln)], riv)
                for g in range(2 * Ln // 16):
                    iv = riv[pl.ds(g * 16, 16)]
                    plsc.addupdate_scatter(cnt, [iv], ones16)
                pltpu.sync_copy(cnt, rcnt_hbm.at[pl.ds(rid * N, N)])
                for g in range(2 * Ln // 16):
                    iv = riv[pl.ds(g * 16, 16)]
                    plsc.store_scatter(cnt, [iv], zero16)

        iotaL = iota16 * DCH

        def scenario(w, carry):
            b = c * BPC + w
            for p in range(2):
                pltpu.sync_copy(
                    src_hbm.at[pl.ds(b * E + p * EH + s * EPH, EPH)], siv)
                pltpu.async_copy(h_hbm.at[siv], rows, sem).wait()

                @plsc.parallel_loop(0, EPH, unroll=2)
                def tr_out(r):
                    for k in range(NDCH):
                        stg[pl.ds((k * EPH + r) * DCH, DCH)] = \
                            rows[r, pl.ds(k * DCH, DCH)]
                dsc = [pltpu.async_copy(
                    stg.at[pl.ds(k * EPH * DCH, EPH * DCH)],
                    planes.at[pl.ds((k * EH + s * EPH) * DCH, EPH * DCH)],
                    sem) for k in range(NDCH)]
                for dd in dsc:
                    dd.wait()
                plsc.subcore_barrier()
                pltpu.sync_copy(planes.at[pl.ds(s * EH * DCH, EH * DCH)],
                                crows.at[pl.ds(p * EH * DCH, EH * DCH)])
                plsc.subcore_barrier()

            pltpu.sync_copy(dst_hbm.at[pl.ds(b * E, E)], div)

            @plsc.parallel_loop(0, N * DCH // 16, unroll=8)
            def zero_agg(i):
                agf[pl.ds(i * 16, 16)] = zero16

            @plsc.parallel_loop(0, E // 16, unroll=2)
            def accum(i):
                dv = div[pl.ds(i * 16, 16)] * DCH
                base = i * (16 * DCH) + iotaL
                for kk in range(DCH):
                    col = plsc.load_gather(crows, [base + kk])
                    plsc.addupdate_scatter(agf, [dv + kk], col)

            for q in range(2):
                pltpu.sync_copy(agf.at[pl.ds(q * N2 * DCH, N2 * DCH)],
                                planes.at[pl.ds(s * N2 * DCH, N2 * DCH)])
                plsc.subcore_barrier()
                dsc = [pltpu.async_copy(
                    planes.at[pl.ds((k * N2 + s * NPH) * DCH, NPH * DCH)],
                    stg.at[pl.ds(k * NPH * DCH, NPH * DCH)],
                    sem) for k in range(NDCH)]
                for dd in dsc:
                    dd.wait()

                @plsc.parallel_loop(0, NPH, unroll=2)
                def tr_in(r):
                    for k in range(NDCH):
                        rows[r, pl.ds(k * DCH, DCH)] = \
                            stg[pl.ds((k * NPH + r) * DCH, DCH)]
                pltpu.sync_copy(
                    rows,
                    agg_hbm.at[pl.ds(b * N + q * N2 + s * NPH, NPH)])
                plsc.subcore_barrier()
            return carry
        lax.fori_loop(0, BPC, scenario, 0)

    if not with_routes:
        def body_nr(h_hbm, src_hbm, dst_hbm, ridx_hbm, agg_hbm,
                    rows, crows, agf, stg, siv, div, planes, sem):
            body(h_hbm, src_hbm, dst_hbm, ridx_hbm, agg_hbm, None,
                 rows, crows, agf, stg, siv, div, planes, sem)
        fn = body_nr
    else:
        fn = body
    return pl.kernel(fn, out_type=out_type, mesh=mesh, scratch_types=scratch,
                     compiler_params=pltpu.CompilerParams(
                         needs_layout_passes=False))


_sc_layer0 = _make_sc_agg(with_routes=True)
_sc_layer = _make_sc_agg(with_routes=False)


def _embed_body(wemb_ref, bemb_ref, h0_ref):
    h0_ref[...] = (wemb_ref[...].astype(jnp.bfloat16).astype(jnp.float32)
                   + bemb_ref[...])


def _layer0_body(h0_ref, agg_ref, w_ref, b_ref, out_ref):
    h = h0_ref[...] + agg_ref[0]
    out_ref[...] = _lrelu(_d(h, w_ref[...]) + b_ref[...]).reshape(1, N, D)


def _layer_body(hp_ref, agg_ref, w_ref, b_ref, out_ref):
    h = hp_ref[0] + agg_ref[0]
    out_ref[...] = _lrelu(_d(h, w_ref[...]) + b_ref[...]).reshape(1, N, D)


def _final_body(h_ref, rcnt_ref,
                ghW1_ref, ghb1_ref, ghW2_ref, ghb2_ref, ghW3_ref, ghb3_ref,
                rhW1_ref, rhb1_ref, rhW2_ref, rhb2_ref, rhW3_ref, rhb3_ref,
                gout_ref, rout_ref):
    f32 = jnp.float32
    h = h_ref[0]
    rc = rcnt_ref[0]
    rmask = (rc > 0).astype(f32)
    counts = jnp.sum(rmask, axis=1, keepdims=True)
    ucnt = jnp.sum(rc, axis=0, keepdims=True)
    used = (ucnt > 0).astype(f32)
    gdesc = _dot_oh(used, h) / 1000.0
    rdesc = _d(rmask, h) / counts

    gh = _lrelu(_d(gdesc, ghW1_ref[...]) + ghb1_ref[...])
    gh = _lrelu(_d(gh, ghW2_ref[...]) + ghb2_ref[...])
    gpred = _d(gh, ghW3_ref[...]) + ghb3_ref[...]

    tiled = jnp.broadcast_to(_lrelu(gdesc), (Rn, D))
    rin = jnp.concatenate([tiled, rdesc], axis=1)
    rh = _lrelu(_d(rin, rhW1_ref[...]) + rhb1_ref[...])
    rh = _lrelu(_d(rh, rhW2_ref[...]) + rhb2_ref[...])
    rpred = _d(rh, rhW3_ref[...]) + rhb3_ref[...]

    gout_ref[...] = gpred.reshape(1, 1, 1)
    rout_ref[...] = rpred.reshape(1, Rn, 1)


def _rep2(shape):
    return pl.BlockSpec(shape, lambda b: (0, 0))


def _layer_call(body, h_spec, h_arr, agg, w, bias):
    return pl.pallas_call(
        body,
        grid=(Bn,),
        in_specs=[h_spec,
                  pl.BlockSpec((1, N, D), lambda b: (b, 0, 0)),
                  _rep2((D, D)), _rep2((1, D))],
        out_specs=pl.BlockSpec((1, N, D), lambda b: (b, 0, 0)),
        out_shape=jax.ShapeDtypeStruct((Bn, N, D), jnp.float32),
        compiler_params=pltpu.CompilerParams(
            dimension_semantics=("arbitrary",)),
    )(h_arr, agg, w, bias)


def kernel(env_rep, batch_route_idxs, Wemb, bemb, gcn_W, gcn_b,
           gh_W1, gh_b1, gh_W2, gh_b2, gh_W3, gh_b3,
           rh_W1, rh_b1, rh_W2, rh_b2, rh_W3, rh_b3):
    f32 = jnp.float32
    src = batch_route_idxs[:, :, 0, :].reshape(Bn, E)
    dst = batch_route_idxs[:, :, 1, :].reshape(Bn, E)
    src1 = src.reshape(Bn * E)
    dst1 = dst.reshape(Bn * E)
    srcoff1 = (src + jnp.arange(Bn, dtype=jnp.int32)[:, None] * N).reshape(Bn * E)
    ridx1 = batch_route_idxs.reshape(Bn * Rn * 2 * Ln)

    h0 = pl.pallas_call(
        _embed_body,
        out_shape=jax.ShapeDtypeStruct((N, D), f32),
    )(Wemb, bemb.reshape(1, D))

    agg0, rcnt_flat = _sc_layer0(h0, src1, dst1, ridx1)
    h = _layer_call(_layer0_body, _rep2((N, D)), h0,
                    agg0.reshape(Bn, N, D), gcn_W[0], gcn_b[0].reshape(1, D))
    for i in range(1, NGL):
        (agg,) = _sc_layer(h.reshape(Bn * N, D), srcoff1, dst1, ridx1)
        h = _layer_call(_layer_body,
                        pl.BlockSpec((1, N, D), lambda b: (b, 0, 0)), h,
                        agg.reshape(Bn, N, D), gcn_W[i],
                        gcn_b[i].reshape(1, D))

    rcnt = rcnt_flat.reshape(Bn, Rn, N)
    gpred, rpred = pl.pallas_call(
        _final_body,
        grid=(Bn,),
        in_specs=[pl.BlockSpec((1, N, D), lambda b: (b, 0, 0)),
                  pl.BlockSpec((1, Rn, N), lambda b: (b, 0, 0)),
                  _rep2((D, D)), _rep2((1, D)),
                  _rep2((D, D)), _rep2((1, D)),
                  _rep2((D, 1)), _rep2((1, 1)),
                  _rep2((2 * D, D)), _rep2((1, D)),
                  _rep2((D, D)), _rep2((1, D)),
                  _rep2((D, 1)), _rep2((1, 1))],
        out_specs=[pl.BlockSpec((1, 1, 1), lambda b: (b, 0, 0)),
                   pl.BlockSpec((1, Rn, 1), lambda b: (b, 0, 0))],
        out_shape=[jax.ShapeDtypeStruct((Bn, 1, 1), f32),
                   jax.ShapeDtypeStruct((Bn, Rn, 1), f32)],
        compiler_params=pltpu.CompilerParams(
            dimension_semantics=("arbitrary",)),
    )(h, rcnt,
      gh_W1, gh_b1.reshape(1, D), gh_W2, gh_b2.reshape(1, D),
      gh_W3, gh_b3.reshape(1, 1),
      rh_W1, rh_b1.reshape(1, D), rh_W2, rh_b2.reshape(1, D),
      rh_W3, rh_b3.reshape(1, 1))
    return gpred.reshape(Bn, 1), rpred


# speedup vs baseline: 1.7532x; 1.0219x over previous
import functools

import jax
import jax.numpy as jnp
from jax import lax
from jax.experimental import pallas as pl
from jax.experimental.pallas import tpu as pltpu
from jax.experimental.pallas import tpu_sc as plsc

N = 2048
D = 256
NGL = 3
Bn, Rn, Ln = 8, 32, 64
E = Rn * Ln
NC, NS = 2, 16
EPT = E // NS
RPW = (Bn * Rn) // (NC * NS)
BPC = Bn // NC


def _lrelu(x):
    return jnp.where(x >= 0, x, 0.01 * x)


def _d(a, b):
    return jnp.dot(a, b, preferred_element_type=jnp.float32)


def _dot_oh(m, x):
    x_hi = x.astype(jnp.bfloat16).astype(jnp.float32)
    return _d(m, x_hi) + _d(m, x - x_hi)


DCH = 16
NDCH = D // DCH
GCH = 128
CPT = NDCH * Bn // (NC * NS)


def _make_sc_agg(with_routes):
    mesh = plsc.VectorSubcoreMesh(core_axis_name="c", subcore_axis_name="s",
                                  num_cores=NC, num_subcores=NS)
    out_type = [jax.ShapeDtypeStruct((Bn * N, D), jnp.float32)]
    if with_routes:
        out_type.append(jax.ShapeDtypeStruct((Bn * Rn * N,), jnp.float32))
    EH = E // 2
    EPH = EH // NS
    N2 = N // 2
    NPH = N2 // NS
    scratch = [
        pltpu.VMEM((EPH, D), jnp.float32),
        pltpu.VMEM((E * DCH,), jnp.float32),
        pltpu.VMEM((N * DCH,), jnp.float32),
        pltpu.VMEM((NDCH * EPH * DCH,), jnp.float32),
        pltpu.VMEM((EPH,), jnp.int32),
        pltpu.VMEM((E,), jnp.int32),
        pltpu.VMEM_SHARED((NDCH * EH * DCH,), jnp.float32),
        pltpu.SemaphoreType.DMA,
        pltpu.SemaphoreType.DMA,
    ]
    if with_routes:
        scratch += [
            pltpu.VMEM((N,), jnp.float32),
            pltpu.VMEM((2 * Ln,), jnp.int32),
        ]

    def body(h_hbm, src_hbm, dst_hbm, ridx_hbm, agg_hbm, rcnt_hbm,
             rows, crows, agf, stg, siv, div, planes, sem, semb,
             cnt=None, riv=None):
        c = lax.axis_index("c")
        s = lax.axis_index("s")
        wid = c * NS + s
        zero16 = jnp.zeros((16,), jnp.float32)
        iota16 = lax.iota(jnp.int32, 16)

        if with_routes:
            ones16 = jnp.ones((16,), jnp.float32)

            @plsc.parallel_loop(0, N // 16, unroll=8)
            def zero_cnt(i):
                cnt[pl.ds(i * 16, 16)] = zero16
            for k in range(RPW):
                rid = wid * RPW + k
                pltpu.sync_copy(ridx_hbm.at[pl.ds(rid * 2 * Ln, 2 * Ln)], riv)
                for g in range(2 * Ln // 16):
                    iv = riv[pl.ds(g * 16, 16)]
                    plsc.addupdate_scatter(cnt, [iv], ones16)
                pltpu.sync_copy(cnt, rcnt_hbm.at[pl.ds(rid * N, N)])
                for g in range(2 * Ln // 16):
                    iv = riv[pl.ds(g * 16, 16)]
                    plsc.store_scatter(cnt, [iv], zero16)

        iotaL = iota16 * DCH

        def scenario(w, carry):
            b = c * BPC + w
            gd = pltpu.async_copy(dst_hbm.at[pl.ds(b * E, E)], div, semb)
            for p in range(2):
                pltpu.sync_copy(
                    src_hbm.at[pl.ds(b * E + p * EH + s * EPH, EPH)], siv)
                pltpu.async_copy(h_hbm.at[siv], rows, sem).wait()

                @plsc.parallel_loop(0, EPH, unroll=2)
                def tr_out(r):
                    for k in range(NDCH):
                        stg[pl.ds((k * EPH + r) * DCH, DCH)] = \
                            rows[r, pl.ds(k * DCH, DCH)]
                dsc = [pltpu.async_copy(
                    stg.at[pl.ds(k * EPH * DCH, EPH * DCH)],
                    planes.at[pl.ds((k * EH + s * EPH) * DCH, EPH * DCH)],
                    sem) for k in range(NDCH)]
                for dd in dsc:
                    dd.wait()
                plsc.subcore_barrier()
                pltpu.sync_copy(planes.at[pl.ds(s * EH * DCH, EH * DCH)],
                                crows.at[pl.ds(p * EH * DCH, EH * DCH)])
                plsc.subcore_barrier()

            gd.wait()

            @plsc.parallel_loop(0, N * DCH // 16, unroll=8)
            def zero_agg(i):
                agf[pl.ds(i * 16, 16)] = zero16

            @plsc.parallel_loop(0, E // 16, unroll=2)
            def accum(i):
                dv = div[pl.ds(i * 16, 16)] * DCH
                base = i * (16 * DCH) + iotaL
                for kk in range(DCH):
                    col = plsc.load_gather(crows, [base + kk])
                    plsc.addupdate_scatter(agf, [dv + kk], col)

            for q in range(2):
                pltpu.sync_copy(agf.at[pl.ds(q * N2 * DCH, N2 * DCH)],
                                planes.at[pl.ds(s * N2 * DCH, N2 * DCH)])
                plsc.subcore_barrier()
                dsc = [pltpu.async_copy(
                    planes.at[pl.ds((k * N2 + s * NPH) * DCH, NPH * DCH)],
                    stg.at[pl.ds(k * NPH * DCH, NPH * DCH)],
                    sem) for k in range(NDCH)]
                for dd in dsc:
                    dd.wait()

                @plsc.parallel_loop(0, NPH, unroll=2)
                def tr_in(r):
                    for k in range(NDCH):
                        rows[r, pl.ds(k * DCH, DCH)] = \
                            stg[pl.ds((k * NPH + r) * DCH, DCH)]
                pltpu.sync_copy(
                    rows,
                    agg_hbm.at[pl.ds(b * N + q * N2 + s * NPH, NPH)])
                plsc.subcore_barrier()
            return carry
        lax.fori_loop(0, BPC, scenario, 0)

    if not with_routes:
        def body_nr(h_hbm, src_hbm, dst_hbm, ridx_hbm, agg_hbm,
                    rows, crows, agf, stg, siv, div, planes, sem, semb):
            body(h_hbm, src_hbm, dst_hbm, ridx_hbm, agg_hbm, None,
                 rows, crows, agf, stg, siv, div, planes, sem, semb)
        fn = body_nr
    else:
        fn = body
    return pl.kernel(fn, out_type=out_type, mesh=mesh, scratch_types=scratch,
                     compiler_params=pltpu.CompilerParams(
                         needs_layout_passes=False))


_sc_layer0 = _make_sc_agg(with_routes=True)
_sc_layer = _make_sc_agg(with_routes=False)


def _embed_body(wemb_ref, bemb_ref, h0_ref):
    h0_ref[...] = (wemb_ref[...].astype(jnp.bfloat16).astype(jnp.float32)
                   + bemb_ref[...])


def _layer0_body(h0_ref, agg_ref, w_ref, b_ref, out_ref):
    h = h0_ref[...] + agg_ref[0]
    out_ref[...] = _lrelu(_d(h, w_ref[...]) + b_ref[...]).reshape(1, N, D)


def _layer_body(hp_ref, agg_ref, w_ref, b_ref, out_ref):
    h = hp_ref[0] + agg_ref[0]
    out_ref[...] = _lrelu(_d(h, w_ref[...]) + b_ref[...]).reshape(1, N, D)


def _final_body(h_ref, rcnt_ref,
                ghW1_ref, ghb1_ref, ghW2_ref, ghb2_ref, ghW3_ref, ghb3_ref,
                rhW1_ref, rhb1_ref, rhW2_ref, rhb2_ref, rhW3_ref, rhb3_ref,
                gout_ref, rout_ref):
    f32 = jnp.float32
    h = h_ref[0]
    rc = rcnt_ref[0]
    rmask = (rc > 0).astype(f32)
    counts = jnp.sum(rmask, axis=1, keepdims=True)
    ucnt = jnp.sum(rc, axis=0, keepdims=True)
    used = (ucnt > 0).astype(f32)
    gdesc = _dot_oh(used, h) / 1000.0
    rdesc = _d(rmask, h) / counts

    gh = _lrelu(_d(gdesc, ghW1_ref[...]) + ghb1_ref[...])
    gh = _lrelu(_d(gh, ghW2_ref[...]) + ghb2_ref[...])
    gpred = _d(gh, ghW3_ref[...]) + ghb3_ref[...]

    tiled = jnp.broadcast_to(_lrelu(gdesc), (Rn, D))
    rin = jnp.concatenate([tiled, rdesc], axis=1)
    rh = _lrelu(_d(rin, rhW1_ref[...]) + rhb1_ref[...])
    rh = _lrelu(_d(rh, rhW2_ref[...]) + rhb2_ref[...])
    rpred = _d(rh, rhW3_ref[...]) + rhb3_ref[...]

    gout_ref[...] = gpred.reshape(1, 1, 1)
    rout_ref[...] = rpred.reshape(1, Rn, 1)


def _rep2(shape):
    return pl.BlockSpec(shape, lambda b: (0, 0))


def _layer_call(body, h_spec, h_arr, agg, w, bias):
    return pl.pallas_call(
        body,
        grid=(Bn,),
        in_specs=[h_spec,
                  pl.BlockSpec((1, N, D), lambda b: (b, 0, 0)),
                  _rep2((D, D)), _rep2((1, D))],
        out_specs=pl.BlockSpec((1, N, D), lambda b: (b, 0, 0)),
        out_shape=jax.ShapeDtypeStruct((Bn, N, D), jnp.float32),
        compiler_params=pltpu.CompilerParams(
            dimension_semantics=("arbitrary",)),
    )(h_arr, agg, w, bias)


def kernel(env_rep, batch_route_idxs, Wemb, bemb, gcn_W, gcn_b,
           gh_W1, gh_b1, gh_W2, gh_b2, gh_W3, gh_b3,
           rh_W1, rh_b1, rh_W2, rh_b2, rh_W3, rh_b3):
    f32 = jnp.float32
    src = batch_route_idxs[:, :, 0, :].reshape(Bn, E)
    dst = batch_route_idxs[:, :, 1, :].reshape(Bn, E)
    src1 = src.reshape(Bn * E)
    dst1 = dst.reshape(Bn * E)
    srcoff1 = (src + jnp.arange(Bn, dtype=jnp.int32)[:, None] * N).reshape(Bn * E)
    ridx1 = batch_route_idxs.reshape(Bn * Rn * 2 * Ln)

    h0 = pl.pallas_call(
        _embed_body,
        out_shape=jax.ShapeDtypeStruct((N, D), f32),
    )(Wemb, bemb.reshape(1, D))

    agg0, rcnt_flat = _sc_layer0(h0, src1, dst1, ridx1)
    h = _layer_call(_layer0_body, _rep2((N, D)), h0,
                    agg0.reshape(Bn, N, D), gcn_W[0], gcn_b[0].reshape(1, D))
    for i in range(1, NGL):
        (agg,) = _sc_layer(h.reshape(Bn * N, D), srcoff1, dst1, ridx1)
        h = _layer_call(_layer_body,
                        pl.BlockSpec((1, N, D), lambda b: (b, 0, 0)), h,
                        agg.reshape(Bn, N, D), gcn_W[i],
                        gcn_b[i].reshape(1, D))

    rcnt = rcnt_flat.reshape(Bn, Rn, N)
    gpred, rpred = pl.pallas_call(
        _final_body,
        grid=(Bn,),
        in_specs=[pl.BlockSpec((1, N, D), lambda b: (b, 0, 0)),
                  pl.BlockSpec((1, Rn, N), lambda b: (b, 0, 0)),
                  _rep2((D, D)), _rep2((1, D)),
                  _rep2((D, D)), _rep2((1, D)),
                  _rep2((D, 1)), _rep2((1, 1)),
                  _rep2((2 * D, D)), _rep2((1, D)),
                  _rep2((D, D)), _rep2((1, D)),
                  _rep2((D, 1)), _rep2((1, 1))],
        out_specs=[pl.BlockSpec((1, 1, 1), lambda b: (b, 0, 0)),
                   pl.BlockSpec((1, Rn, 1), lambda b: (b, 0, 0))],
        out_shape=[jax.ShapeDtypeStruct((Bn, 1, 1), f32),
                   jax.ShapeDtypeStruct((Bn, Rn, 1), f32)],
        compiler_params=pltpu.CompilerParams(
            dimension_semantics=("arbitrary",)),
    )(h, rcnt,
      gh_W1, gh_b1.reshape(1, D), gh_W2, gh_b2.reshape(1, D),
      gh_W3, gh_b3.reshape(1, 1),
      rh_W1, rh_b1.reshape(1, D), rh_W2, rh_b2.reshape(1, D),
      rh_W3, rh_b3.reshape(1, 1))
    return gpred.reshape(Bn, 1), rpred
